# baseline (speedup 1.0000x reference)
import functools

import jax
import jax.numpy as jnp
from jax import lax
from jax.experimental import pallas as pl
from jax.experimental.pallas import tpu as pltpu
from jax.experimental.pallas import tpu_sc as plsc

VOCAB = 100000
EMBED = 64
BATCH = 4096
HIST = 50
CHUNK = 128
NW = 32
SB = BATCH
STOT = SB * HIST
SCPW = STOT // (CHUNK * NW)


def _sc_gather(x3d, table):
    mesh = plsc.VectorSubcoreMesh(core_axis_name="c", subcore_axis_name="s")

    @functools.partial(
        pl.kernel,
        mesh=mesh,
        compiler_params=pltpu.CompilerParams(use_tc_tiling_on_sc=False),
        out_type=jax.ShapeDtypeStruct((STOT, EMBED), jnp.float32),
        scratch_types=[
            pltpu.VMEM((SCPW, CHUNK), jnp.int32),
            pltpu.VMEM((4, CHUNK, EMBED), jnp.float32),
            [pltpu.SemaphoreType.DMA] * 4,
            [pltpu.SemaphoreType.DMA] * 4,
        ],
    )
    def k(idx_hbm, table_hbm, out_hbm, idx_v, bufs, gsems, osems):
        wid = lax.axis_index("s") * 2 + lax.axis_index("c")
        row0 = wid * SCPW
        pltpu.sync_copy(idx_hbm.at[wid], idx_v)

        def start_gather(j, k4):
            pltpu.async_copy(table_hbm.at[idx_v.at[j]], bufs.at[k4], gsems[k4])

        def start_out(j, k4):
            pltpu.async_copy(bufs.at[k4],
                             out_hbm.at[pl.ds((row0 + j) * CHUNK, CHUNK)],
                             osems[k4])

        def wait_gather(k4):
            pltpu.make_async_copy(out_hbm.at[pl.ds(0, CHUNK)], bufs.at[k4],
                                  gsems[k4]).wait()

        def wait_out(k4):
            pltpu.make_async_copy(bufs.at[k4], out_hbm.at[pl.ds(0, CHUNK)],
                                  osems[k4]).wait()

        for k4 in range(4):
            start_gather(k4, k4)

        def body(g, carry):
            c = 4 * g
            for k4 in range(4):
                wait_gather(k4)
                start_out(c + k4, k4)
            for k4 in range(4):
                wait_out(k4)
                nxt = c + 4 + k4

                @pl.when(nxt < SCPW)
                def _():
                    start_gather(nxt, k4)
            return carry

        lax.fori_loop(0, SCPW // 4, body, 0)
        for k4 in range(SCPW % 4):
            wait_gather(k4)
            start_out(SCPW - (SCPW % 4) + k4, k4)
        for k4 in range(SCPW % 4):
            wait_out(k4)

    return k(x3d, table)


def _mlp_block(x4_ref, w1_ref, b1_ref, w2_ref, b2_ref, out_ref):
    x4 = x4_ref[...]
    flat = jnp.transpose(x4, (0, 2, 1, 3)).reshape(x4.shape[0] * 8, 25 * 128)
    fb = flat.astype(jnp.bfloat16)
    h = jnp.dot(fb, w1_ref[...], preferred_element_type=jnp.float32,
                precision=lax.Precision.DEFAULT)
    h = jnp.maximum(h + b1_ref[...], 0.0)
    o = jnp.dot(h.astype(jnp.bfloat16), w2_ref[...],
                preferred_element_type=jnp.float32,
                precision=lax.Precision.DEFAULT)
    out_ref[...] = jnp.maximum(o + b2_ref[...], 0.0)


def _tc_mlp(x4, W1b, b1, W2b, b2):
    BB = 512
    grid = (SB // BB,)
    return pl.pallas_call(
        _mlp_block,
        grid=grid,
        in_specs=[
            pl.BlockSpec((BB // 8, 25, 8, 128), lambda i: (i, 0, 0, 0)),
            pl.BlockSpec((HIST * EMBED, 1024), lambda i: (0, 0)),
            pl.BlockSpec((1, 1024), lambda i: (0, 0)),
            pl.BlockSpec((1024, 512), lambda i: (0, 0)),
            pl.BlockSpec((1, 512), lambda i: (0, 0)),
        ],
        out_specs=pl.BlockSpec((BB, 512), lambda i: (i, 0)),
        out_shape=jax.ShapeDtypeStruct((SB, 512), jnp.float32),
    )(x4, W1b, b1, W2b, b2)


def kernel(x, table, W1, b1, W2, b2):
    W1b = W1.astype(jnp.bfloat16)
    W2b = W2.astype(jnp.bfloat16)
    perm = (jnp.arange(SB * HIST, dtype=jnp.int32)
            .reshape(SB // 8, 8, 25, 2).transpose(0, 2, 1, 3).reshape(-1))
    xp = x.reshape(-1)[perm]
    rows = _sc_gather(xp.reshape(NW, SCPW, CHUNK), table)
    x4 = rows.reshape(SB // 8, 25, 8, 128)
    return _tc_mlp(x4, W1b, b1.reshape(1, -1), W2b, b2.reshape(1, -1))


# speedup vs baseline: 1.2316x; 1.2316x over previous
import functools

import jax
import jax.numpy as jnp
from jax import lax
from jax.experimental import pallas as pl
from jax.experimental.pallas import tpu as pltpu
from jax.experimental.pallas import tpu_sc as plsc

VOCAB = 100000
EMBED = 64
BATCH = 4096
HIST = 50
CHUNK = 128
NW = 32
SB = BATCH
STOT = SB * HIST
SCPW = STOT // (CHUNK * NW)


def _sc_gather(x3d, table):
    mesh = plsc.VectorSubcoreMesh(core_axis_name="c", subcore_axis_name="s")

    @functools.partial(
        pl.kernel,
        mesh=mesh,
        compiler_params=pltpu.CompilerParams(use_tc_tiling_on_sc=False),
        out_type=jax.ShapeDtypeStruct((STOT, EMBED), jnp.float32),
        scratch_types=[
            pltpu.VMEM((SCPW, CHUNK), jnp.int32),
            pltpu.VMEM((4, CHUNK, EMBED), jnp.float32),
            [pltpu.SemaphoreType.DMA] * 4,
            [pltpu.SemaphoreType.DMA] * 4,
        ],
    )
    def k(idx_hbm, table_hbm, out_hbm, idx_v, bufs, gsems, osems):
        wid = lax.axis_index("s") * 2 + lax.axis_index("c")
        row0 = wid * SCPW
        pltpu.sync_copy(idx_hbm.at[wid], idx_v)

        def start_gather(j, k4):
            pltpu.async_copy(table_hbm.at[idx_v.at[j]], bufs.at[k4], gsems[k4])

        def start_out(j, k4):
            pltpu.async_copy(bufs.at[k4],
                             out_hbm.at[pl.ds((row0 + j) * CHUNK, CHUNK)],
                             osems[k4])

        def wait_gather(k4):
            pltpu.make_async_copy(out_hbm.at[pl.ds(0, CHUNK)], bufs.at[k4],
                                  gsems[k4]).wait()

        def wait_out(k4):
            pltpu.make_async_copy(bufs.at[k4], out_hbm.at[pl.ds(0, CHUNK)],
                                  osems[k4]).wait()

        for k4 in range(4):
            start_gather(k4, k4)

        def body(g, carry):
            c = 4 * g
            for k4 in range(4):
                wait_gather(k4)
                start_out(c + k4, k4)
            for k4 in range(4):
                wait_out(k4)
                nxt = c + 4 + k4

                @pl.when(nxt < SCPW)
                def _():
                    start_gather(nxt, k4)
            return carry

        lax.fori_loop(0, SCPW // 4, body, 0)
        for k4 in range(SCPW % 4):
            wait_gather(k4)
            start_out(SCPW - (SCPW % 4) + k4, k4)
        for k4 in range(SCPW % 4):
            wait_out(k4)

    return k(x3d, table)


def _mlp_block(flat_ref, w1_ref, b1_ref, w2_ref, b2_ref, out_ref):
    fb = flat_ref[...].astype(jnp.bfloat16)
    h = jnp.dot(fb, w1_ref[...], preferred_element_type=jnp.float32,
                precision=lax.Precision.DEFAULT)
    h = jnp.maximum(h + b1_ref[...], 0.0)
    o = jnp.dot(h.astype(jnp.bfloat16), w2_ref[...],
                preferred_element_type=jnp.float32,
                precision=lax.Precision.DEFAULT)
    out_ref[...] = jnp.maximum(o + b2_ref[...], 0.0)


def _tc_mlp(flat, W1b, b1, W2b, b2):
    BB = 512
    grid = (SB // BB,)
    return pl.pallas_call(
        _mlp_block,
        grid=grid,
        in_specs=[
            pl.BlockSpec((BB, HIST * EMBED), lambda i: (i, 0)),
            pl.BlockSpec((HIST * EMBED, 1024), lambda i: (0, 0)),
            pl.BlockSpec((1, 1024), lambda i: (0, 0)),
            pl.BlockSpec((1024, 512), lambda i: (0, 0)),
            pl.BlockSpec((1, 512), lambda i: (0, 0)),
        ],
        out_specs=pl.BlockSpec((BB, 512), lambda i: (i, 0)),
        out_shape=jax.ShapeDtypeStruct((SB, 512), jnp.float32),
    )(flat, W1b, b1, W2b, b2)


def kernel(x, table, W1, b1, W2, b2):
    W1b = W1.astype(jnp.bfloat16)
    W2b = W2.astype(jnp.bfloat16)
    rows = _sc_gather(x.reshape(NW, SCPW, CHUNK), table)
    flat = rows.reshape(SB, HIST * EMBED)
    return _tc_mlp(flat, W1b, b1.reshape(1, -1), W2b, b2.reshape(1, -1))
